# baseline (device time: 30345 ns/iter reference)
import jax
import jax.numpy as jnp
from jax import lax
from jax.experimental import pallas as pl
from jax.experimental.pallas import tpu as pltpu

N_DEV = 8
M_BLK = 512
K_BLK = 512
N_OUT = 2048


def kernel(x, w_mat, scale_x, scale_w):
    m_total, k_blk = x.shape
    assert (m_total, k_blk) == (N_DEV * M_BLK, K_BLK)

    def body(x_ref, w_ref, sx_ref, sw_ref, out_ref,
             xg_ref, acc_ref, send_sems, recv_sems):
        my = lax.axis_index("i")

        barrier_sem = pltpu.get_barrier_semaphore()
        for d in range(1, N_DEV):
            tgt = lax.rem(my + d, N_DEV)
            pl.semaphore_signal(
                barrier_sem, inc=1,
                device_id=(tgt,), device_id_type=pl.DeviceIdType.MESH,
            )
        pl.semaphore_wait(barrier_sem, N_DEV - 1)

        for d in range(1, N_DEV):
            tgt = lax.rem(my + d, N_DEV)
            rdma = pltpu.make_async_remote_copy(
                src_ref=x_ref.at[pl.ds(tgt * M_BLK, M_BLK), :],
                dst_ref=xg_ref.at[my],
                send_sem=send_sems.at[d - 1],
                recv_sem=recv_sems.at[my],
                device_id=(tgt,),
                device_id_type=pl.DeviceIdType.MESH,
            )
            rdma.start()

        acc_ref[...] = jnp.dot(
            x_ref[pl.ds(my * M_BLK, M_BLK), :],
            w_ref[pl.ds(my * K_BLK, K_BLK), :],
            preferred_element_type=jnp.int32,
        )

        for d in range(1, N_DEV):
            src = lax.rem(my + N_DEV - d, N_DEV)
            recv = pltpu.make_async_remote_copy(
                src_ref=x_ref.at[pl.ds(0, M_BLK), :],
                dst_ref=xg_ref.at[src],
                send_sem=send_sems.at[d - 1],
                recv_sem=recv_sems.at[src],
                device_id=(src,),
                device_id_type=pl.DeviceIdType.MESH,
            )
            recv.wait_recv()
            acc_ref[...] += jnp.dot(
                xg_ref[src],
                w_ref[pl.ds(src * K_BLK, K_BLK), :],
                preferred_element_type=jnp.int32,
            )

        for d in range(1, N_DEV):
            tgt = lax.rem(my + d, N_DEV)
            send = pltpu.make_async_remote_copy(
                src_ref=x_ref.at[pl.ds(tgt * M_BLK, M_BLK), :],
                dst_ref=xg_ref.at[my],
                send_sem=send_sems.at[d - 1],
                recv_sem=recv_sems.at[my],
                device_id=(tgt,),
                device_id_type=pl.DeviceIdType.MESH,
            )
            send.wait_send()

        scale = sx_ref[0] * sw_ref[0]
        out_ref[...] = jnp.maximum(
            acc_ref[...].astype(jnp.float32) * scale, 0.0
        )

    return pl.pallas_call(
        body,
        out_shape=jax.ShapeDtypeStruct((M_BLK, N_OUT), jnp.float32),
        in_specs=[
            pl.BlockSpec(memory_space=pltpu.VMEM),
            pl.BlockSpec(memory_space=pltpu.VMEM),
            pl.BlockSpec(memory_space=pltpu.SMEM),
            pl.BlockSpec(memory_space=pltpu.SMEM),
        ],
        out_specs=pl.BlockSpec(memory_space=pltpu.VMEM),
        scratch_shapes=[
            pltpu.VMEM((N_DEV, M_BLK, K_BLK), jnp.int8),
            pltpu.VMEM((M_BLK, N_OUT), jnp.int32),
            pltpu.SemaphoreType.DMA((N_DEV - 1,)),
            pltpu.SemaphoreType.DMA((N_DEV,)),
        ],
        compiler_params=pltpu.CompilerParams(collective_id=0),
    )(x, w_mat, scale_x, scale_w)
